# baseline (speedup 1.0000x reference)
import functools

import jax
import jax.numpy as jnp
from jax import lax
from jax.experimental import pallas as pl
from jax.experimental.pallas import tpu as pltpu
from jax.experimental.pallas import tpu_sc as plsc

B = 16384
DP, DC, DS = 32, 16, 24
DO = DP + DC + DS
DOP = 128
PV = 100001
NC, NS = 2, 16
NW = NC * NS
BW = B // NW

NP = 781
PQ_PAD = 25008
TAIL0 = NP * 128

_mesh = plsc.VectorSubcoreMesh(core_axis_name="c", subcore_axis_name="s")


@functools.partial(
    pl.kernel,
    out_type=jax.ShapeDtypeStruct((PQ_PAD, 128), jnp.float32),
    mesh=_mesh,
    scratch_types=[
        pltpu.VMEM((DP, 128), jnp.float32),
        pltpu.VMEM((32, 128), jnp.float32),
    ],
    compiler_params=pltpu.CompilerParams(use_tc_tiling_on_sc=True,
                                         needs_layout_passes=False),
)
def _transpose_pack(ptab_t_hbm, tail_hbm, out_hbm, pan_v, tpan_v):
    wid = lax.axis_index("s") * NC + lax.axis_index("c")

    nk = NP // NW + 1

    def panel_step(k, carry):
        cid = wid + k * NW

        @pl.when(cid < NP)
        def _():
            iota = lax.iota(jnp.int32, 16)
            iota_d4 = lax.shift_right_logical(iota, 2)
            iam3 = (iota & 3) * 32
            p0 = cid * 128
            pltpu.sync_copy(ptab_t_hbm.at[:, pl.ds(p0, 128)], pan_v)
            for j0 in range(8):
                q_vec = iota_d4 + (j0 * 4)
                for c in range(DP):
                    l_vec = iam3 + c
                    v = pan_v[c, pl.ds(j0 * 16, 16)]
                    plsc.store_scatter(tpan_v, [q_vec, l_vec], v)
            pltpu.sync_copy(tpan_v, out_hbm.at[pl.ds(cid * 32, 32), :])

        return carry

    lax.fori_loop(0, nk, panel_step, 0)

    @pl.when(wid == NW - 1)
    def _():
        pltpu.sync_copy(tail_hbm, tpan_v.at[pl.ds(0, 16), :])
        pltpu.sync_copy(tpan_v.at[pl.ds(0, 16), :],
                        out_hbm.at[pl.ds(NP * 32, 16), :])


@functools.partial(
    pl.kernel,
    out_type=jax.ShapeDtypeStruct((B, DOP), jnp.float32),
    mesh=_mesh,
    scratch_types=[
        pltpu.VMEM((BW,), jnp.int32),
        pltpu.VMEM((BW,), jnp.int32),
        pltpu.VMEM((BW,), jnp.int32),
        pltpu.VMEM((BW,), jnp.int32),
        pltpu.VMEM((BW, DC), jnp.float32),
        pltpu.VMEM((BW, DS), jnp.float32),
        pltpu.VMEM((BW, DOP), jnp.float32),
        pltpu.SemaphoreType.DMA,
    ],
    compiler_params=pltpu.CompilerParams(use_tc_tiling_on_sc=False,
                                         needs_layout_passes=False),
)
def _sc_kernel(pidq_hbm, pidm_hbm, cid_hbm, sid_hbm, ptabq_hbm, ctab_hbm,
               stab_hbm, out_hbm, pidq_v, pidm_v, cidx_v, sidx_v, cat_v,
               sub_v, row_v, sem):
    wid = lax.axis_index("s") * NC + lax.axis_index("c")
    base = wid * BW
    pltpu.sync_copy(pidq_hbm.at[pl.ds(base, BW)], pidq_v)
    pltpu.sync_copy(pidm_hbm.at[pl.ds(base, BW)], pidm_v)
    pltpu.sync_copy(cid_hbm.at[pl.ds(base, BW)], cidx_v)
    pltpu.sync_copy(sid_hbm.at[pl.ds(base, BW)], sidx_v)
    cp1 = pltpu.async_copy(ptabq_hbm.at[pidq_v], row_v, sem)
    cp2 = pltpu.async_copy(ctab_hbm.at[cidx_v], cat_v, sem)
    cp3 = pltpu.async_copy(stab_hbm.at[sidx_v], sub_v, sem)
    cp1.wait()
    cp2.wait()
    cp3.wait()

    @plsc.parallel_loop(0, BW, step=16)
    def _extract(h):
        iota = lax.iota(jnp.int32, 16)
        rows = iota + h
        rm = pidm_v[pl.ds(h, 16)]
        for j in range(DP):
            src_lane = rm + j
            v = plsc.load_gather(row_v, [rows, src_lane])
            plsc.store_scatter(row_v, [rows, iota * 0 + j], v)

    @plsc.parallel_loop(0, BW, unroll=8)
    def _assemble(r):
        row_v[r, pl.ds(32, 16)] = cat_v[r, pl.ds(0, 16)]
        row_v[r, pl.ds(48, 16)] = sub_v[r, pl.ds(0, 16)]
        row_v[r, pl.ds(56, 16)] = sub_v[r, pl.ds(8, 16)]

    pltpu.sync_copy(row_v, out_hbm.at[pl.ds(base, BW)])


def kernel(product_id, stratbuy_domain_desc, mge_main_cat_desc,
           product_table, category_table, subcategory_table):
    pid = product_id.astype(jnp.int32)
    pidq = lax.shift_right_logical(pid, 2)
    pidm = (pid & 3) * 32
    ptab_t = product_table.T
    tail = jnp.pad(product_table[TAIL0:], ((0, 64 - (PV - TAIL0)), (0, 0)))
    tail16 = tail.reshape(16, 128)
    ptabq = _transpose_pack(ptab_t, tail16)
    out = _sc_kernel(
        pidq, pidm,
        stratbuy_domain_desc.astype(jnp.int32),
        mge_main_cat_desc.astype(jnp.int32),
        ptabq, category_table, subcategory_table)
    return out[:, :DO]


# speedup vs baseline: 1.3377x; 1.3377x over previous
import functools

import jax
import jax.numpy as jnp
from jax import lax
from jax.experimental import pallas as pl
from jax.experimental.pallas import tpu as pltpu
from jax.experimental.pallas import tpu_sc as plsc

B = 16384
DP, DC, DS = 32, 16, 24
DO = DP + DC + DS
DOP = 128
PV = 100001
NC, NS = 2, 16
NW = NC * NS
BW = B // NW

NP = 781
PQ_PAD = 25008
TAIL0 = NP * 128

_mesh = plsc.VectorSubcoreMesh(core_axis_name="c", subcore_axis_name="s")


@functools.partial(
    pl.kernel,
    out_type=jax.ShapeDtypeStruct((PQ_PAD, 128), jnp.float32),
    mesh=_mesh,
    scratch_types=[
        pltpu.VMEM((DP, 128), jnp.float32),
        pltpu.VMEM((DP, 128), jnp.float32),
        pltpu.VMEM((32, 128), jnp.float32),
        pltpu.VMEM((32, 128), jnp.float32),
        pltpu.SemaphoreType.DMA,
        pltpu.SemaphoreType.DMA,
        pltpu.SemaphoreType.DMA,
        pltpu.SemaphoreType.DMA,
    ],
    compiler_params=pltpu.CompilerParams(use_tc_tiling_on_sc=True,
                                         needs_layout_passes=False),
)
def _transpose_pack(ptab_t_hbm, tail_hbm, out_hbm, pan0, pan1, tpan0, tpan1,
                    si0, si1, so0, so1):
    wid = lax.axis_index("s") * NC + lax.axis_index("c")
    pans, tpans = (pan0, pan1), (tpan0, tpan1)
    sis, sos = (si0, si1), (so0, so1)
    nk = NP // NW + 1

    def cid_of(k):
        return jnp.minimum(wid + k * NW, NP - 1)

    def issue_in(k, par):
        pltpu.async_copy(ptab_t_hbm.at[:, pl.ds(cid_of(k) * 128, 128)],
                         pans[par], sis[par])

    def transpose(pan, tpan):
        iota = lax.iota(jnp.int32, 16)
        iota_d4 = lax.shift_right_logical(iota, 2)
        iam3 = (iota & 3) * 32
        for j0 in range(8):
            q_vec = iota_d4 + (j0 * 4)
            for c in range(DP):
                l_vec = iam3 + c
                plsc.store_scatter(tpan, [q_vec, l_vec],
                                   pan[c, pl.ds(j0 * 16, 16)])

    def step(k, par):
        issue_in(k + 1, 1 - par)
        pltpu.make_async_copy(ptab_t_hbm.at[:, pl.ds(0, 128)],
                              pans[par], sis[par]).wait()

        @pl.when(k >= 2)
        def _():
            pltpu.make_async_copy(tpans[par], out_hbm.at[pl.ds(0, 32), :],
                                  sos[par]).wait()

        transpose(pans[par], tpans[par])
        pltpu.async_copy(tpans[par], out_hbm.at[pl.ds(cid_of(k) * 32, 32), :],
                         sos[par])

    issue_in(0, 0)

    def pair(kk, carry):
        step(2 * kk, 0)
        step(2 * kk + 1, 1)
        return carry

    lax.fori_loop(0, (nk - 1) // 2, pair, 0)
    k_last = nk - 1
    pltpu.make_async_copy(ptab_t_hbm.at[:, pl.ds(0, 128)],
                          pans[0], sis[0]).wait()
    pltpu.make_async_copy(tpans[0], out_hbm.at[pl.ds(0, 32), :],
                          sos[0]).wait()
    transpose(pans[0], tpans[0])
    pltpu.async_copy(tpans[0], out_hbm.at[pl.ds(cid_of(k_last) * 32, 32), :],
                     sos[0])
    pltpu.make_async_copy(tpans[1], out_hbm.at[pl.ds(0, 32), :], sos[1]).wait()
    pltpu.make_async_copy(tpans[0], out_hbm.at[pl.ds(0, 32), :], sos[0]).wait()

    @pl.when(wid == NW - 1)
    def _():
        pltpu.sync_copy(tail_hbm, tpan0.at[pl.ds(0, 16), :])
        pltpu.sync_copy(tpan0.at[pl.ds(0, 16), :],
                        out_hbm.at[pl.ds(NP * 32, 16), :])


@functools.partial(
    pl.kernel,
    out_type=jax.ShapeDtypeStruct((B, DOP), jnp.float32),
    mesh=_mesh,
    scratch_types=[
        pltpu.VMEM((BW,), jnp.int32),
        pltpu.VMEM((BW,), jnp.int32),
        pltpu.VMEM((BW,), jnp.int32),
        pltpu.VMEM((BW, DP), jnp.float32),
        pltpu.VMEM((BW, DC), jnp.float32),
        pltpu.VMEM((BW, DS), jnp.float32),
        pltpu.VMEM((BW, DOP), jnp.float32),
        pltpu.SemaphoreType.DMA,
    ],
    compiler_params=pltpu.CompilerParams(use_tc_tiling_on_sc=False,
                                         needs_layout_passes=False),
)
def _sc_kernel(pid_hbm, cid_hbm, sid_hbm, ptab_hbm, ctab_hbm,
               stab_hbm, out_hbm, pidx_v, cidx_v, sidx_v, prod_v, cat_v,
               sub_v, row_v, sem):
    wid = lax.axis_index("s") * NC + lax.axis_index("c")
    base = wid * BW
    pltpu.sync_copy(pid_hbm.at[pl.ds(base, BW)], pidx_v)
    pltpu.sync_copy(cid_hbm.at[pl.ds(base, BW)], cidx_v)
    pltpu.sync_copy(sid_hbm.at[pl.ds(base, BW)], sidx_v)
    cp1 = pltpu.async_copy(ptab_hbm.at[pidx_v], prod_v, sem)
    cp2 = pltpu.async_copy(ctab_hbm.at[cidx_v], cat_v, sem)
    cp3 = pltpu.async_copy(stab_hbm.at[sidx_v], sub_v, sem)
    cp1.wait()
    cp2.wait()
    cp3.wait()

    @plsc.parallel_loop(0, BW, unroll=8)
    def _assemble(r):
        row_v[r, pl.ds(0, 16)] = prod_v[r, pl.ds(0, 16)]
        row_v[r, pl.ds(16, 16)] = prod_v[r, pl.ds(16, 16)]
        row_v[r, pl.ds(32, 16)] = cat_v[r, pl.ds(0, 16)]
        row_v[r, pl.ds(48, 16)] = sub_v[r, pl.ds(0, 16)]
        row_v[r, pl.ds(56, 16)] = sub_v[r, pl.ds(8, 16)]

    pltpu.sync_copy(row_v, out_hbm.at[pl.ds(base, BW)])


def kernel(product_id, stratbuy_domain_desc, mge_main_cat_desc,
           product_table, category_table, subcategory_table):
    pid = product_id.astype(jnp.int32)
    ptab_t = product_table.T
    tail = jnp.pad(product_table[TAIL0:], ((0, 64 - (PV - TAIL0)), (0, 0)))
    tail16 = tail.reshape(16, 128)
    ptab32 = _transpose_pack(ptab_t, tail16).reshape(PQ_PAD * 4, DP)
    out = _sc_kernel(
        pid,
        stratbuy_domain_desc.astype(jnp.int32),
        mge_main_cat_desc.astype(jnp.int32),
        ptab32, category_table, subcategory_table)
    return out[:, :DO]


# speedup vs baseline: 1.4512x; 1.0849x over previous
import functools

import jax
import jax.numpy as jnp
from jax import lax
from jax.experimental import pallas as pl
from jax.experimental.pallas import tpu as pltpu
from jax.experimental.pallas import tpu_sc as plsc

B = 16384
DP, DC, DS = 32, 16, 24
DO = DP + DC + DS
DOP = 128
PV = 100001
NC, NS = 2, 16
NW = NC * NS
BW = B // NW

NP = 781
PQ_PAD = 25008
TAIL0 = NP * 128

_mesh = plsc.VectorSubcoreMesh(core_axis_name="c", subcore_axis_name="s")


@functools.partial(
    pl.kernel,
    out_type=jax.ShapeDtypeStruct((PQ_PAD, 128), jnp.float32),
    mesh=_mesh,
    scratch_types=[
        pltpu.VMEM((DP, 128), jnp.float32),
        pltpu.VMEM((DP, 128), jnp.float32),
        pltpu.VMEM((32, 128), jnp.float32),
        pltpu.VMEM((32, 128), jnp.float32),
        pltpu.SemaphoreType.DMA,
        pltpu.SemaphoreType.DMA,
        pltpu.SemaphoreType.DMA,
        pltpu.SemaphoreType.DMA,
    ],
    compiler_params=pltpu.CompilerParams(use_tc_tiling_on_sc=True,
                                         needs_layout_passes=False),
)
def _transpose_pack(ptab_t_hbm, tail_hbm, out_hbm, pan0, pan1, tpan0, tpan1,
                    si0, si1, so0, so1):
    wid = lax.axis_index("s") * NC + lax.axis_index("c")
    pans, tpans = (pan0, pan1), (tpan0, tpan1)
    sis, sos = (si0, si1), (so0, so1)
    nk = NP // NW + 1

    def cid_of(k):
        return jnp.minimum(wid + k * NW, NP - 1)

    def issue_in(k, par):
        pltpu.async_copy(ptab_t_hbm.at[:, pl.ds(cid_of(k) * 128, 128)],
                         pans[par], sis[par])

    def transpose(pan, tpan):
        @plsc.parallel_loop(0, 8, unroll=4)
        def _group(j0):
            iota = lax.iota(jnp.int32, 16)
            q_vec = lax.shift_right_logical(iota, 2) + j0 * 4
            iam3 = (iota & 3) * 32
            for c in range(DP):
                l_vec = iam3 + c
                plsc.store_scatter(tpan, [q_vec, l_vec],
                                   pan[c, pl.ds(j0 * 16, 16)])

    def step(k, par):
        issue_in(k + 1, 1 - par)
        pltpu.make_async_copy(ptab_t_hbm.at[:, pl.ds(0, 128)],
                              pans[par], sis[par]).wait()

        @pl.when(k >= 2)
        def _():
            pltpu.make_async_copy(tpans[par], out_hbm.at[pl.ds(0, 32), :],
                                  sos[par]).wait()

        transpose(pans[par], tpans[par])
        pltpu.async_copy(tpans[par], out_hbm.at[pl.ds(cid_of(k) * 32, 32), :],
                         sos[par])

    issue_in(0, 0)

    def pair(kk, carry):
        step(2 * kk, 0)
        step(2 * kk + 1, 1)
        return carry

    lax.fori_loop(0, (nk - 1) // 2, pair, 0)
    k_last = nk - 1
    pltpu.make_async_copy(ptab_t_hbm.at[:, pl.ds(0, 128)],
                          pans[0], sis[0]).wait()
    pltpu.make_async_copy(tpans[0], out_hbm.at[pl.ds(0, 32), :],
                          sos[0]).wait()
    transpose(pans[0], tpans[0])
    pltpu.async_copy(tpans[0], out_hbm.at[pl.ds(cid_of(k_last) * 32, 32), :],
                     sos[0])
    pltpu.make_async_copy(tpans[1], out_hbm.at[pl.ds(0, 32), :], sos[1]).wait()
    pltpu.make_async_copy(tpans[0], out_hbm.at[pl.ds(0, 32), :], sos[0]).wait()

    @pl.when(wid == NW - 1)
    def _():
        pltpu.sync_copy(tail_hbm, tpan0.at[pl.ds(0, 16), :])
        pltpu.sync_copy(tpan0.at[pl.ds(0, 16), :],
                        out_hbm.at[pl.ds(NP * 32, 16), :])


@functools.partial(
    pl.kernel,
    out_type=jax.ShapeDtypeStruct((B, DOP), jnp.float32),
    mesh=_mesh,
    scratch_types=[
        pltpu.VMEM((BW,), jnp.int32),
        pltpu.VMEM((BW,), jnp.int32),
        pltpu.VMEM((BW,), jnp.int32),
        pltpu.VMEM((BW, DP), jnp.float32),
        pltpu.VMEM((BW, DC), jnp.float32),
        pltpu.VMEM((BW, DS), jnp.float32),
        pltpu.VMEM((BW, DOP), jnp.float32),
        pltpu.SemaphoreType.DMA,
    ],
    compiler_params=pltpu.CompilerParams(use_tc_tiling_on_sc=False,
                                         needs_layout_passes=False),
)
def _sc_kernel(pid_hbm, cid_hbm, sid_hbm, ptab_hbm, ctab_hbm,
               stab_hbm, out_hbm, pidx_v, cidx_v, sidx_v, prod_v, cat_v,
               sub_v, row_v, sem):
    wid = lax.axis_index("s") * NC + lax.axis_index("c")
    base = wid * BW
    pltpu.sync_copy(pid_hbm.at[pl.ds(base, BW)], pidx_v)
    pltpu.sync_copy(cid_hbm.at[pl.ds(base, BW)], cidx_v)
    pltpu.sync_copy(sid_hbm.at[pl.ds(base, BW)], sidx_v)
    cp1 = pltpu.async_copy(ptab_hbm.at[pidx_v], prod_v, sem)
    cp2 = pltpu.async_copy(ctab_hbm.at[cidx_v], cat_v, sem)
    cp3 = pltpu.async_copy(stab_hbm.at[sidx_v], sub_v, sem)
    cp1.wait()
    cp2.wait()
    cp3.wait()

    @plsc.parallel_loop(0, BW, unroll=8)
    def _assemble(r):
        row_v[r, pl.ds(0, 16)] = prod_v[r, pl.ds(0, 16)]
        row_v[r, pl.ds(16, 16)] = prod_v[r, pl.ds(16, 16)]
        row_v[r, pl.ds(32, 16)] = cat_v[r, pl.ds(0, 16)]
        row_v[r, pl.ds(48, 16)] = sub_v[r, pl.ds(0, 16)]
        row_v[r, pl.ds(56, 16)] = sub_v[r, pl.ds(8, 16)]

    pltpu.sync_copy(row_v, out_hbm.at[pl.ds(base, BW)])


def kernel(product_id, stratbuy_domain_desc, mge_main_cat_desc,
           product_table, category_table, subcategory_table):
    pid = product_id.astype(jnp.int32)
    ptab_t = product_table.T
    tail = jnp.pad(product_table[TAIL0:], ((0, 64 - (PV - TAIL0)), (0, 0)))
    tail16 = tail.reshape(16, 128)
    ptab32 = _transpose_pack(ptab_t, tail16).reshape(PQ_PAD * 4, DP)
    out = _sc_kernel(
        pid,
        stratbuy_domain_desc.astype(jnp.int32),
        mge_main_cat_desc.astype(jnp.int32),
        ptab32, category_table, subcategory_table)
    return out[:, :DO]
